# Initial kernel scaffold; baseline (speedup 1.0000x reference)
import jax
import jax.numpy as jnp
from jax.experimental import pallas as pl


def kernel(x, table):
    raise NotImplementedError("write your pallas kernel here")


# speedup vs baseline: 5.0098x; 5.0098x over previous
import functools

import jax
import jax.numpy as jnp
from jax import lax
from jax.experimental import pallas as pl
from jax.experimental.pallas import tpu as pltpu
from jax.experimental.pallas import tpu_sc as plsc

BATCH = 16384
HIST = 200
EMBED = 64
TOTAL = BATCH * HIST

IDX_W = 128
NB = 8
CHUNK = NB * IDX_W


def _build():
    info = plsc.get_sparse_core_info()
    nc, ns = info.num_cores, info.num_subcores
    nw = nc * ns
    per_w = TOTAL // nw
    steps = per_w // CHUNK
    rows_per_w = per_w // IDX_W

    mesh = plsc.VectorSubcoreMesh(core_axis_name="c", subcore_axis_name="s")

    @functools.partial(
        pl.kernel,
        mesh=mesh,
        out_type=jax.ShapeDtypeStruct((TOTAL, EMBED), jnp.float32),
        scratch_types=[
            pltpu.VMEM((NB, IDX_W), jnp.int32),
            pltpu.VMEM((CHUNK, EMBED), jnp.float32),
            pltpu.SemaphoreType.DMA,
        ],
        compiler_params=pltpu.CompilerParams(use_tc_tiling_on_sc=False),
    )
    def gather_kernel(x_hbm, table_hbm, out_hbm, idx_v, rows_v, sem):
        wid = lax.axis_index("s") * nc + lax.axis_index("c")
        row0 = wid * rows_per_w
        base0 = wid * per_w

        def body(g, carry):
            pltpu.sync_copy(x_hbm.at[pl.ds(row0 + g * NB, NB)], idx_v)
            copies = []
            for j in range(NB):
                copies.append(
                    pltpu.async_copy(
                        table_hbm.at[idx_v.at[j]],
                        rows_v.at[pl.ds(j * IDX_W, IDX_W)],
                        sem,
                    )
                )
            for c in copies:
                c.wait()
            pltpu.sync_copy(rows_v, out_hbm.at[pl.ds(base0 + g * CHUNK, CHUNK)])
            return carry

        lax.fori_loop(0, steps, body, 0)

    return gather_kernel


_GATHER = _build()


@jax.jit
def kernel(x, table):
    x_flat = x.reshape(TOTAL // IDX_W, IDX_W).astype(jnp.int32)
    out = _GATHER(x_flat, table)
    return out.reshape(BATCH, HIST, EMBED)


# speedup vs baseline: 5.2030x; 1.0386x over previous
import functools

import jax
import jax.numpy as jnp
from jax import lax
from jax.experimental import pallas as pl
from jax.experimental.pallas import tpu as pltpu
from jax.experimental.pallas import tpu_sc as plsc

BATCH = 16384
HIST = 200
EMBED = 64
TOTAL = BATCH * HIST

IDX_W = 128
NB = 5
CHUNK = NB * IDX_W


def _build():
    info = plsc.get_sparse_core_info()
    nc, ns = info.num_cores, info.num_subcores
    nw = nc * ns
    per_w = TOTAL // nw
    steps = per_w // CHUNK

    mesh = plsc.VectorSubcoreMesh(core_axis_name="c", subcore_axis_name="s")

    @functools.partial(
        pl.kernel,
        mesh=mesh,
        out_type=jax.ShapeDtypeStruct((TOTAL, EMBED), jnp.float32),
        scratch_types=[
            pltpu.VMEM((CHUNK,), jnp.int32),
            pltpu.VMEM((CHUNK,), jnp.int32),
            pltpu.VMEM((CHUNK, EMBED), jnp.float32),
            pltpu.VMEM((CHUNK, EMBED), jnp.float32),
            pltpu.SemaphoreType.DMA,
            pltpu.SemaphoreType.DMA,
            pltpu.SemaphoreType.DMA,
            pltpu.SemaphoreType.DMA,
            pltpu.SemaphoreType.DMA,
            pltpu.SemaphoreType.DMA,
        ],
        compiler_params=pltpu.CompilerParams(use_tc_tiling_on_sc=False),
    )
    def gather_kernel(x_hbm, table_hbm, out_hbm,
                      idx0, idx1, rows0, rows1,
                      asem0, asem1, gsem0, gsem1, wsem0, wsem1):
        wid = lax.axis_index("s") * nc + lax.axis_index("c")
        base0 = wid * per_w

        idx_b = (idx0, idx1)
        rows_b = (rows0, rows1)
        asem = (asem0, asem1)
        gsem = (gsem0, gsem1)
        wsem = (wsem0, wsem1)

        def fire_idx(t, b):
            pltpu.async_copy(
                x_hbm.at[pl.ds(base0 + t * CHUNK, CHUNK)], idx_b[b], asem[b])

        def wait_idx(b):
            pltpu.make_async_copy(
                x_hbm.at[pl.ds(base0, CHUNK)], idx_b[b], asem[b]).wait()

        def fire_gather(b):
            for j in range(NB):
                pltpu.async_copy(
                    table_hbm.at[idx_b[b].at[pl.ds(j * IDX_W, IDX_W)]],
                    rows_b[b].at[pl.ds(j * IDX_W, IDX_W)],
                    gsem[b],
                )

        def wait_gather(b):
            pltpu.make_async_copy(
                out_hbm.at[pl.ds(base0, CHUNK)], rows_b[b], gsem[b]).wait()

        def fire_write(t, b):
            pltpu.async_copy(
                rows_b[b], out_hbm.at[pl.ds(base0 + t * CHUNK, CHUNK)],
                wsem[b])

        def wait_write(b):
            pltpu.make_async_copy(
                rows_b[b], out_hbm.at[pl.ds(base0, CHUNK)], wsem[b]).wait()

        def slot(t, b, first=False, last=False):
            if not first:
                wait_write(b)
            wait_idx(b)
            fire_gather(b)
            if not first:
                wait_gather(1 - b)
                fire_write(t - 1, 1 - b)
            if not last:
                fire_idx(t + 1, 1 - b)

        fire_idx(0, 0)
        wait_idx(0)
        fire_gather(0)
        fire_idx(1, 1)
        wait_idx(1)
        fire_gather(1)
        wait_gather(0)
        fire_write(0, 0)
        fire_idx(2, 0)

        def pair(s, carry):
            t = 2 * s
            slot(t, 0)
            slot(t + 1, 1)
            return carry

        lax.fori_loop(1, steps // 2 - 1, pair, 0)

        slot(steps - 2, 0)
        slot(steps - 1, 1, last=True)

        wait_gather(1)
        fire_write(steps - 1, 1)
        wait_write(0)
        wait_write(1)

    return gather_kernel


_GATHER = _build()


@jax.jit
def kernel(x, table):
    x_flat = x.reshape(TOTAL).astype(jnp.int32)
    out = _GATHER(x_flat, table)
    return out.reshape(BATCH, HIST, EMBED)
